# baseline (speedup 1.0000x reference)
import functools

import jax
import jax.numpy as jnp
from jax import lax
from jax.experimental import pallas as pl
from jax.experimental.pallas import tpu as pltpu
from jax.experimental.pallas import tpu_sc as plsc

_i32 = jnp.int32
_MIN_NORM = 1e-15
_EPS = 1e-7
_N = 10000
_E = 320000
_D = 128

_CHUNK = 120
_CORES = 2
_SUBCORES = 16
_TILES = _CORES * _SUBCORES
_CHUNKS = 88
_NPAD = 10112
_BR = 400


def _arcosh(t):
    return jnp.log(t + jnp.sqrt(jnp.clip(t * t - 1.0, 0.0, None)))


def _tangent_body(x_ref, xt_ref):
    xb = x_ref[...]
    col = lax.broadcasted_iota(jnp.int32, xb.shape, 1)
    y = jnp.where(col == 0, 0.0, xb)
    n = jnp.maximum(jnp.sqrt(jnp.sum(y * y, axis=1, keepdims=True)), _MIN_NORM)
    e = jnp.exp(n)
    rest = (0.5 * (e - 1.0 / e)) * y / n
    r2 = jnp.sum(rest * rest, axis=1, keepdims=True)
    first = jnp.sqrt(jnp.maximum(1.0 + r2, _EPS))
    yn = jnp.maximum(jnp.sqrt(r2), _MIN_NORM)
    theta = jnp.maximum(first, 1.0 + _EPS)
    xt_ref[...] = jnp.where(col == 0, 0.0, (_arcosh(theta) / yn) * rest)


def _combine_body(eps_ref, x_ref, xt_ref, p_ref, w1_ref, b1_ref, w2_ref, b2_ref,
                  o_ref):
    eps = eps_ref[0]
    xb = x_ref[...]
    xt = xt_ref[...]
    sup = p_ref[0] + p_ref[1]
    col = lax.broadcasted_iota(jnp.int32, xb.shape, 1)

    def c0(a):
        return jnp.sum(jnp.where(col == 0, a, 0.0), axis=1, keepdims=True)

    def mdot(a, b):
        return jnp.sum(a * b, axis=1, keepdims=True) - 2.0 * c0(a) * c0(b)

    def expmap0(u):
        y = jnp.where(col == 0, 0.0, u)
        n = jnp.maximum(jnp.sqrt(jnp.sum(y * y, axis=1, keepdims=True)), _MIN_NORM)
        e = jnp.exp(n)
        rest = (0.5 * (e - 1.0 / e)) * y / n
        r2 = jnp.sum(rest * rest, axis=1, keepdims=True)
        first = jnp.sqrt(jnp.maximum(1.0 + r2, _EPS))
        return jnp.where(col == 0, first, rest)

    def logmap0(xh):
        y = jnp.where(col == 0, 0.0, xh)
        yn = jnp.maximum(jnp.sqrt(jnp.sum(y * y, axis=1, keepdims=True)), _MIN_NORM)
        theta = jnp.maximum(c0(xh), 1.0 + _EPS)
        return jnp.where(col == 0, 0.0, (_arcosh(theta) / yn) * y)

    x_h = expmap0(xb)
    out1 = expmap0(sup)
    log_out = logmap0(out1)

    prod = mdot(x_h, out1)
    theta_d = jnp.maximum(-prod, 1.0 + _EPS)
    sq = jnp.minimum(_arcosh(theta_d) ** 2, 50.0)
    dist = jnp.sqrt(sq)

    def logmap_pair(a, b):
        xy = jnp.minimum(mdot(a, b) + 1.0, -_EPS) - 1.0
        u = b + xy * a
        normu = jnp.maximum(jnp.sqrt(jnp.maximum(mdot(u, u), _EPS)), _MIN_NORM)
        return dist * u / normu

    logxy = logmap_pair(x_h, out1)
    logyx = logmap_pair(out1, x_h)
    alpha = mdot(logxy, xt) / jnp.maximum(sq, _MIN_NORM)
    res = xt - alpha * (logxy + logyx)
    ux = jnp.sum(jnp.where(col == 0, 0.0, out1 * res), axis=1, keepdims=True)
    first_pt = ux / jnp.maximum(c0(out1), _EPS)
    pt = jnp.where(col == 0, first_pt, res)

    out2 = expmap0(log_out + (1.0 + eps) * pt)
    h = jnp.maximum(jnp.dot(out2, w1_ref[...],
                            preferred_element_type=jnp.float32) + b1_ref[...], 0.0)
    o_ref[...] = jnp.dot(h, w2_ref[...],
                         preferred_element_type=jnp.float32) + b2_ref[...]


def _sc_segment_sum(xt, src_blocks, dst_blocks, zeros):
    mesh = plsc.VectorSubcoreMesh(core_axis_name="c", subcore_axis_name="s")
    grp = 8

    @functools.partial(
        pl.kernel,
        mesh=mesh,
        out_type=jax.ShapeDtypeStruct((_CORES, _NPAD, _D), jnp.float32),
        scratch_types=[
            pltpu.VMEM((grp, _CHUNK), jnp.int32),
            pltpu.VMEM((grp, _CHUNK), jnp.int32),
            pltpu.VMEM((_CHUNK, _D), jnp.float32),
            pltpu.VMEM((_CHUNK, _D), jnp.float32),
            pltpu.VMEM((_CHUNK, _D), jnp.float32),
            pltpu.VMEM_SHARED((_NPAD, _D), jnp.float32),
            pltpu.SemaphoreType.DMA,
            pltpu.SemaphoreType.DMA,
            pltpu.SemaphoreType.DMA,
            pltpu.SemaphoreType.DMA,
            pltpu.SemaphoreType.DMA,
            pltpu.SemaphoreType.DMA,
        ],
    )
    def body(xt_hbm, src_hbm, dst_hbm, zeros_hbm, out_hbm,
             sidx, didx, buf0, buf1, buf2, acc,
             gs0, gs1, gs2, ss0, ss1, ss2):
        cid = lax.axis_index("c")
        sid = lax.axis_index("s")
        rz = _NPAD // _SUBCORES
        pltpu.sync_copy(zeros_hbm, acc.at[pl.ds(sid * rz, rz)])
        plsc.subcore_barrier()

        bufs = (buf0, buf1, buf2)
        gsems = (gs0, gs1, gs2)
        ssems = (ss0, ss1, ss2)

        def gather(k, p):
            return pltpu.make_async_copy(xt_hbm.at[sidx.at[_i32(k)]],
                                         bufs[p], gsems[p])

        def scat(k, p):
            return pltpu.make_async_copy(bufs[p], acc.at[didx.at[_i32(k)]],
                                         ssems[p])

        def step(i, carry):
            j0 = pl.multiple_of(i * jnp.int32(grp), grp)
            pltpu.sync_copy(src_hbm.at[cid, sid, pl.ds(j0, grp)], sidx)
            pltpu.sync_copy(dst_hbm.at[cid, sid, pl.ds(j0, grp)], didx)
            gather(0, 0).start()
            gather(1, 1).start()
            for k in range(grp):
                p = k % 3
                gather(k, p).wait()
                scat(k, p).start(add=True)
                if k + 2 < grp:
                    pw = (k + 2) % 3
                    if k >= 1:
                        scat(k - 1, pw).wait()
                    gather(k + 2, pw).start()
            scat(grp - 3, (grp - 3) % 3).wait()
            scat(grp - 2, (grp - 2) % 3).wait()
            scat(grp - 1, (grp - 1) % 3).wait()
            return carry

        lax.fori_loop(jnp.int32(0), jnp.int32(_CHUNKS // grp), step,
                      jnp.int32(0))
        plsc.subcore_barrier()
        pltpu.sync_copy(acc.at[pl.ds(sid * rz, rz)],
                        out_hbm.at[cid, pl.ds(sid * rz, rz)])

    return body(xt, src_blocks, dst_blocks, zeros)


def kernel(x, edge_index, eps, W1, b1, W2, b2):
    x = x.astype(jnp.float32)
    dst = edge_index[0].astype(jnp.int32)
    src = edge_index[1].astype(jnp.int32)

    xt = pl.pallas_call(
        _tangent_body,
        grid=(_N // _BR,),
        in_specs=[pl.BlockSpec((_BR, _D), lambda i: (_i32(i), _i32(0)))],
        out_specs=pl.BlockSpec((_BR, _D), lambda i: (_i32(i), _i32(0))),
        out_shape=jax.ShapeDtypeStruct((_N, _D), jnp.float32),
    )(x)

    pad_n = _TILES * _CHUNKS * _CHUNK - _E
    pad_src = jnp.arange(pad_n, dtype=jnp.int32) % _N
    src_p = jnp.concatenate([src, pad_src]).reshape(
        _CORES, _SUBCORES, _CHUNKS, _CHUNK)
    pad_dst = _N + (jnp.arange(pad_n, dtype=jnp.int32) % (_NPAD - _N))
    dst_p = jnp.concatenate([dst, pad_dst]).reshape(
        _CORES, _SUBCORES, _CHUNKS, _CHUNK)
    zeros = jnp.zeros((_NPAD // _SUBCORES, _D), jnp.float32)

    partials = _sc_segment_sum(xt, src_p, dst_p, zeros)

    eps_arr = jnp.reshape(eps.astype(jnp.float32), (1,))
    out = pl.pallas_call(
        _combine_body,
        grid=(_N // _BR,),
        in_specs=[
            pl.BlockSpec((1,), lambda i: (_i32(0),), memory_space=pltpu.SMEM),
            pl.BlockSpec((_BR, _D), lambda i: (_i32(i), _i32(0))),
            pl.BlockSpec((_BR, _D), lambda i: (_i32(i), _i32(0))),
            pl.BlockSpec((_CORES, _BR, _D), lambda i: (_i32(0), _i32(i), _i32(0))),
            pl.BlockSpec((_D, _D), lambda i: (_i32(0), _i32(0))),
            pl.BlockSpec((1, _D), lambda i: (_i32(0), _i32(0))),
            pl.BlockSpec((_D, _D), lambda i: (_i32(0), _i32(0))),
            pl.BlockSpec((1, _D), lambda i: (_i32(0), _i32(0))),
        ],
        out_specs=pl.BlockSpec((_BR, _D), lambda i: (_i32(i), _i32(0))),
        out_shape=jax.ShapeDtypeStruct((_N, _D), jnp.float32),
    )(eps_arr, x, xt, partials, W1, jnp.reshape(b1, (1, _D)),
      W2, jnp.reshape(b2, (1, _D)))
    return out


# speedup vs baseline: 1.0292x; 1.0292x over previous
import functools

import jax
import jax.numpy as jnp
from jax import lax
from jax.experimental import pallas as pl
from jax.experimental.pallas import tpu as pltpu
from jax.experimental.pallas import tpu_sc as plsc

_i32 = jnp.int32
_MIN_NORM = 1e-15
_EPS = 1e-7
_N = 10000
_E = 320000
_D = 128

_CHUNK = 128
_CORES = 2
_SUBCORES = 16
_TILES = _CORES * _SUBCORES
_CHUNKS = 80
_NPAD = 10240
_BR = 400


def _arcosh(t):
    return jnp.log(t + jnp.sqrt(jnp.clip(t * t - 1.0, 0.0, None)))


def _tangent_body(x_ref, xt_ref):
    xb = x_ref[...]
    col = lax.broadcasted_iota(jnp.int32, xb.shape, 1)
    y = jnp.where(col == 0, 0.0, xb)
    n = jnp.maximum(jnp.sqrt(jnp.sum(y * y, axis=1, keepdims=True)), _MIN_NORM)
    e = jnp.exp(n)
    rest = (0.5 * (e - 1.0 / e)) * y / n
    r2 = jnp.sum(rest * rest, axis=1, keepdims=True)
    first = jnp.sqrt(jnp.maximum(1.0 + r2, _EPS))
    yn = jnp.maximum(jnp.sqrt(r2), _MIN_NORM)
    theta = jnp.maximum(first, 1.0 + _EPS)
    xt_ref[...] = jnp.where(col == 0, 0.0, (_arcosh(theta) / yn) * rest)


def _combine_body(eps_ref, x_ref, xt_ref, p_ref, w1_ref, b1_ref, w2_ref, b2_ref,
                  o_ref):
    eps = eps_ref[0]
    xb = x_ref[...]
    xt = xt_ref[...]
    sup = p_ref[0] + p_ref[1]
    col = lax.broadcasted_iota(jnp.int32, xb.shape, 1)

    def c0(a):
        return jnp.sum(jnp.where(col == 0, a, 0.0), axis=1, keepdims=True)

    def mdot(a, b):
        return jnp.sum(a * b, axis=1, keepdims=True) - 2.0 * c0(a) * c0(b)

    def expmap0(u):
        y = jnp.where(col == 0, 0.0, u)
        n = jnp.maximum(jnp.sqrt(jnp.sum(y * y, axis=1, keepdims=True)), _MIN_NORM)
        e = jnp.exp(n)
        rest = (0.5 * (e - 1.0 / e)) * y / n
        r2 = jnp.sum(rest * rest, axis=1, keepdims=True)
        first = jnp.sqrt(jnp.maximum(1.0 + r2, _EPS))
        return jnp.where(col == 0, first, rest)

    def logmap0(xh):
        y = jnp.where(col == 0, 0.0, xh)
        yn = jnp.maximum(jnp.sqrt(jnp.sum(y * y, axis=1, keepdims=True)), _MIN_NORM)
        theta = jnp.maximum(c0(xh), 1.0 + _EPS)
        return jnp.where(col == 0, 0.0, (_arcosh(theta) / yn) * y)

    x_h = expmap0(xb)
    out1 = expmap0(sup)
    log_out = logmap0(out1)

    prod = mdot(x_h, out1)
    theta_d = jnp.maximum(-prod, 1.0 + _EPS)
    sq = jnp.minimum(_arcosh(theta_d) ** 2, 50.0)
    dist = jnp.sqrt(sq)

    def logmap_pair(a, b):
        xy = jnp.minimum(mdot(a, b) + 1.0, -_EPS) - 1.0
        u = b + xy * a
        normu = jnp.maximum(jnp.sqrt(jnp.maximum(mdot(u, u), _EPS)), _MIN_NORM)
        return dist * u / normu

    logxy = logmap_pair(x_h, out1)
    logyx = logmap_pair(out1, x_h)
    alpha = mdot(logxy, xt) / jnp.maximum(sq, _MIN_NORM)
    res = xt - alpha * (logxy + logyx)
    ux = jnp.sum(jnp.where(col == 0, 0.0, out1 * res), axis=1, keepdims=True)
    first_pt = ux / jnp.maximum(c0(out1), _EPS)
    pt = jnp.where(col == 0, first_pt, res)

    out2 = expmap0(log_out + (1.0 + eps) * pt)
    h = jnp.maximum(jnp.dot(out2, w1_ref[...],
                            preferred_element_type=jnp.float32) + b1_ref[...], 0.0)
    o_ref[...] = jnp.dot(h, w2_ref[...],
                         preferred_element_type=jnp.float32) + b2_ref[...]


def _sc_segment_sum(xt, src_blocks, dst_blocks, zeros):
    mesh = plsc.VectorSubcoreMesh(core_axis_name="c", subcore_axis_name="s")
    grp = 8

    @functools.partial(
        pl.kernel,
        mesh=mesh,
        out_type=jax.ShapeDtypeStruct((_CORES, _NPAD, _D), jnp.float32),
        scratch_types=[
            pltpu.VMEM((grp, _CHUNK), jnp.int32),
            pltpu.VMEM((grp, _CHUNK), jnp.int32),
            pltpu.VMEM((_CHUNK, _D), jnp.float32),
            pltpu.VMEM((_CHUNK, _D), jnp.float32),
            pltpu.VMEM_SHARED((_NPAD, _D), jnp.float32),
            pltpu.SemaphoreType.DMA,
            pltpu.SemaphoreType.DMA,
        ],
    )
    def body(xt_hbm, src_hbm, dst_hbm, zeros_hbm, out_hbm,
             sidx, didx, buf0, buf1, acc, sem0, sem1):
        cid = lax.axis_index("c")
        sid = lax.axis_index("s")
        rz = _NPAD // _SUBCORES
        pltpu.sync_copy(zeros_hbm, acc.at[pl.ds(sid * rz, rz)])
        plsc.subcore_barrier()

        bufs = (buf0, buf1)
        sems = (sem0, sem1)

        def gather(k, p):
            return pltpu.make_async_copy(xt_hbm.at[sidx.at[_i32(k)]],
                                         bufs[p], sems[p])

        def step(i, carry):
            j0 = pl.multiple_of(i * jnp.int32(grp), grp)
            pltpu.sync_copy(src_hbm.at[cid, sid, pl.ds(j0, grp)], sidx)
            pltpu.sync_copy(dst_hbm.at[cid, sid, pl.ds(j0, grp)], didx)
            gather(0, 0).start()
            gather(1, 1).start()
            for k in range(grp):
                p = k % 2
                gather(k, p).wait()
                pltpu.sync_copy(bufs[p], acc.at[didx.at[_i32(k)]], add=True)
                if k + 2 < grp:
                    gather(k + 2, p).start()
            return carry

        lax.fori_loop(jnp.int32(0), jnp.int32(_CHUNKS // grp), step,
                      jnp.int32(0))
        plsc.subcore_barrier()
        pltpu.sync_copy(acc.at[pl.ds(sid * rz, rz)],
                        out_hbm.at[cid, pl.ds(sid * rz, rz)])

    return body(xt, src_blocks, dst_blocks, zeros)


def kernel(x, edge_index, eps, W1, b1, W2, b2):
    x = x.astype(jnp.float32)
    dst = edge_index[0].astype(jnp.int32)
    src = edge_index[1].astype(jnp.int32)

    xt = pl.pallas_call(
        _tangent_body,
        grid=(_N // _BR,),
        in_specs=[pl.BlockSpec((_BR, _D), lambda i: (_i32(i), _i32(0)))],
        out_specs=pl.BlockSpec((_BR, _D), lambda i: (_i32(i), _i32(0))),
        out_shape=jax.ShapeDtypeStruct((_N, _D), jnp.float32),
    )(x)

    pad_n = _TILES * _CHUNKS * _CHUNK - _E
    pad_src = jnp.arange(pad_n, dtype=jnp.int32) % _N
    src_p = jnp.concatenate([src, pad_src]).reshape(
        _CORES, _SUBCORES, _CHUNKS, _CHUNK)
    pad_dst = _N + (jnp.arange(pad_n, dtype=jnp.int32) % (_NPAD - _N))
    dst_p = jnp.concatenate([dst, pad_dst]).reshape(
        _CORES, _SUBCORES, _CHUNKS, _CHUNK)
    zeros = jnp.zeros((_NPAD // _SUBCORES, _D), jnp.float32)

    partials = _sc_segment_sum(xt, src_p, dst_p, zeros)

    eps_arr = jnp.reshape(eps.astype(jnp.float32), (1,))
    out = pl.pallas_call(
        _combine_body,
        grid=(_N // _BR,),
        in_specs=[
            pl.BlockSpec((1,), lambda i: (_i32(0),), memory_space=pltpu.SMEM),
            pl.BlockSpec((_BR, _D), lambda i: (_i32(i), _i32(0))),
            pl.BlockSpec((_BR, _D), lambda i: (_i32(i), _i32(0))),
            pl.BlockSpec((_CORES, _BR, _D), lambda i: (_i32(0), _i32(i), _i32(0))),
            pl.BlockSpec((_D, _D), lambda i: (_i32(0), _i32(0))),
            pl.BlockSpec((1, _D), lambda i: (_i32(0), _i32(0))),
            pl.BlockSpec((_D, _D), lambda i: (_i32(0), _i32(0))),
            pl.BlockSpec((1, _D), lambda i: (_i32(0), _i32(0))),
        ],
        out_specs=pl.BlockSpec((_BR, _D), lambda i: (_i32(i), _i32(0))),
        out_shape=jax.ShapeDtypeStruct((_N, _D), jnp.float32),
    )(eps_arr, x, xt, partials, W1, jnp.reshape(b1, (1, _D)),
      W2, jnp.reshape(b2, (1, _D)))
    return out


# speedup vs baseline: 1.0815x; 1.0507x over previous
import functools

import jax
import jax.numpy as jnp
from jax import lax
from jax.experimental import pallas as pl
from jax.experimental.pallas import tpu as pltpu
from jax.experimental.pallas import tpu_sc as plsc

_i32 = jnp.int32
_MIN_NORM = 1e-15
_EPS = 1e-7
_N = 10000
_E = 320000
_D = 128

_CHUNK = 128
_CORES = 2
_SUBCORES = 16
_TILES = _CORES * _SUBCORES
_CHUNKS = 80
_NPAD = 10240
_BR = 1000


def _arcosh(t):
    return jnp.log(t + jnp.sqrt(jnp.clip(t * t - 1.0, 0.0, None)))


def _tangent_body(x_ref, xt_ref):
    xb = x_ref[...]
    col = lax.broadcasted_iota(jnp.int32, xb.shape, 1)
    y = jnp.where(col == 0, 0.0, xb)
    n = jnp.maximum(jnp.sqrt(jnp.sum(y * y, axis=1, keepdims=True)), _MIN_NORM)
    e = jnp.exp(n)
    rest = (0.5 * (e - 1.0 / e)) * y / n
    r2 = jnp.sum(rest * rest, axis=1, keepdims=True)
    first = jnp.sqrt(jnp.maximum(1.0 + r2, _EPS))
    yn = jnp.maximum(jnp.sqrt(r2), _MIN_NORM)
    theta = jnp.maximum(first, 1.0 + _EPS)
    xt_ref[...] = jnp.where(col == 0, 0.0, (_arcosh(theta) / yn) * rest)


def _combine_body(eps_ref, x_ref, xt_ref, p_ref, w1_ref, b1_ref, w2_ref, b2_ref,
                  o_ref):
    eps = eps_ref[0]
    xb = x_ref[...]
    xt = xt_ref[...]
    sup = p_ref[0] + p_ref[1]
    col = lax.broadcasted_iota(jnp.int32, xb.shape, 1)

    def c0(a):
        return jnp.sum(jnp.where(col == 0, a, 0.0), axis=1, keepdims=True)

    def mdot(a, b):
        return jnp.sum(a * b, axis=1, keepdims=True) - 2.0 * c0(a) * c0(b)

    def expmap0(u):
        y = jnp.where(col == 0, 0.0, u)
        n = jnp.maximum(jnp.sqrt(jnp.sum(y * y, axis=1, keepdims=True)), _MIN_NORM)
        e = jnp.exp(n)
        rest = (0.5 * (e - 1.0 / e)) * y / n
        r2 = jnp.sum(rest * rest, axis=1, keepdims=True)
        first = jnp.sqrt(jnp.maximum(1.0 + r2, _EPS))
        return jnp.where(col == 0, first, rest)

    def logmap0(xh):
        y = jnp.where(col == 0, 0.0, xh)
        yn = jnp.maximum(jnp.sqrt(jnp.sum(y * y, axis=1, keepdims=True)), _MIN_NORM)
        theta = jnp.maximum(c0(xh), 1.0 + _EPS)
        return jnp.where(col == 0, 0.0, (_arcosh(theta) / yn) * y)

    x_h = expmap0(xb)
    out1 = expmap0(sup)
    log_out = logmap0(out1)

    prod = mdot(x_h, out1)
    theta_d = jnp.maximum(-prod, 1.0 + _EPS)
    sq = jnp.minimum(_arcosh(theta_d) ** 2, 50.0)
    dist = jnp.sqrt(sq)

    def logmap_pair(a, b):
        xy = jnp.minimum(mdot(a, b) + 1.0, -_EPS) - 1.0
        u = b + xy * a
        normu = jnp.maximum(jnp.sqrt(jnp.maximum(mdot(u, u), _EPS)), _MIN_NORM)
        return dist * u / normu

    logxy = logmap_pair(x_h, out1)
    logyx = logmap_pair(out1, x_h)
    alpha = mdot(logxy, xt) / jnp.maximum(sq, _MIN_NORM)
    res = xt - alpha * (logxy + logyx)
    ux = jnp.sum(jnp.where(col == 0, 0.0, out1 * res), axis=1, keepdims=True)
    first_pt = ux / jnp.maximum(c0(out1), _EPS)
    pt = jnp.where(col == 0, first_pt, res)

    out2 = expmap0(log_out + (1.0 + eps) * pt)
    h = jnp.maximum(jnp.dot(out2, w1_ref[...],
                            preferred_element_type=jnp.float32) + b1_ref[...], 0.0)
    o_ref[...] = jnp.dot(h, w2_ref[...],
                         preferred_element_type=jnp.float32) + b2_ref[...]


def _sc_segment_sum(xt, src_blocks, dst_blocks, zeros):
    mesh = plsc.VectorSubcoreMesh(core_axis_name="c", subcore_axis_name="s")
    grp = 8

    @functools.partial(
        pl.kernel,
        mesh=mesh,
        out_type=jax.ShapeDtypeStruct((_CORES, _NPAD, _D), jnp.float32),
        scratch_types=[
            pltpu.VMEM((grp, _CHUNK), jnp.int32),
            pltpu.VMEM((grp, _CHUNK), jnp.int32),
            pltpu.VMEM((_CHUNK, _D), jnp.float32),
            pltpu.VMEM((_CHUNK, _D), jnp.float32),
            pltpu.VMEM_SHARED((_NPAD, _D), jnp.float32),
            pltpu.SemaphoreType.DMA,
            pltpu.SemaphoreType.DMA,
        ],
    )
    def body(xt_hbm, src_hbm, dst_hbm, zeros_hbm, out_hbm,
             sidx, didx, buf0, buf1, acc, sem0, sem1):
        cid = lax.axis_index("c")
        sid = lax.axis_index("s")
        rz = _NPAD // _SUBCORES
        pltpu.sync_copy(zeros_hbm, acc.at[pl.ds(sid * rz, rz)])
        plsc.subcore_barrier()

        bufs = (buf0, buf1)
        sems = (sem0, sem1)

        def gather(k, p):
            return pltpu.make_async_copy(xt_hbm.at[sidx.at[_i32(k)]],
                                         bufs[p], sems[p])

        def step(i, carry):
            j0 = pl.multiple_of(i * jnp.int32(grp), grp)
            pltpu.sync_copy(src_hbm.at[cid, sid, pl.ds(j0, grp)], sidx)
            pltpu.sync_copy(dst_hbm.at[cid, sid, pl.ds(j0, grp)], didx)
            gather(0, 0).start()
            gather(1, 1).start()
            for k in range(grp):
                p = k % 2
                gather(k, p).wait()
                pltpu.sync_copy(bufs[p], acc.at[didx.at[_i32(k)]], add=True)
                if k + 2 < grp:
                    gather(k + 2, p).start()
            return carry

        lax.fori_loop(jnp.int32(0), jnp.int32(_CHUNKS // grp), step,
                      jnp.int32(0))
        plsc.subcore_barrier()
        pltpu.sync_copy(acc.at[pl.ds(sid * rz, rz)],
                        out_hbm.at[cid, pl.ds(sid * rz, rz)])

    return body(xt, src_blocks, dst_blocks, zeros)


def kernel(x, edge_index, eps, W1, b1, W2, b2):
    x = x.astype(jnp.float32)
    dst = edge_index[0].astype(jnp.int32)
    src = edge_index[1].astype(jnp.int32)

    xt = pl.pallas_call(
        _tangent_body,
        grid=(_N // _BR,),
        in_specs=[pl.BlockSpec((_BR, _D), lambda i: (_i32(i), _i32(0)))],
        out_specs=pl.BlockSpec((_BR, _D), lambda i: (_i32(i), _i32(0))),
        out_shape=jax.ShapeDtypeStruct((_N, _D), jnp.float32),
    )(x)

    pad_n = _TILES * _CHUNKS * _CHUNK - _E
    pad_src = jnp.arange(pad_n, dtype=jnp.int32) % _N
    src_p = jnp.concatenate([src, pad_src]).reshape(
        _CORES, _SUBCORES, _CHUNKS, _CHUNK)
    pad_dst = _N + (jnp.arange(pad_n, dtype=jnp.int32) % (_NPAD - _N))
    dst_p = jnp.concatenate([dst, pad_dst]).reshape(
        _CORES, _SUBCORES, _CHUNKS, _CHUNK)
    zeros = jnp.zeros((_NPAD // _SUBCORES, _D), jnp.float32)

    partials = _sc_segment_sum(xt, src_p, dst_p, zeros)

    eps_arr = jnp.reshape(eps.astype(jnp.float32), (1,))
    out = pl.pallas_call(
        _combine_body,
        grid=(_N // _BR,),
        in_specs=[
            pl.BlockSpec((1,), lambda i: (_i32(0),), memory_space=pltpu.SMEM),
            pl.BlockSpec((_BR, _D), lambda i: (_i32(i), _i32(0))),
            pl.BlockSpec((_BR, _D), lambda i: (_i32(i), _i32(0))),
            pl.BlockSpec((_CORES, _BR, _D), lambda i: (_i32(0), _i32(i), _i32(0))),
            pl.BlockSpec((_D, _D), lambda i: (_i32(0), _i32(0))),
            pl.BlockSpec((1, _D), lambda i: (_i32(0), _i32(0))),
            pl.BlockSpec((_D, _D), lambda i: (_i32(0), _i32(0))),
            pl.BlockSpec((1, _D), lambda i: (_i32(0), _i32(0))),
        ],
        out_specs=pl.BlockSpec((_BR, _D), lambda i: (_i32(i), _i32(0))),
        out_shape=jax.ShapeDtypeStruct((_N, _D), jnp.float32),
    )(eps_arr, x, xt, partials, W1, jnp.reshape(b1, (1, _D)),
      W2, jnp.reshape(b2, (1, _D)))
    return out


# speedup vs baseline: 1.0844x; 1.0027x over previous
import functools

import jax
import jax.numpy as jnp
from jax import lax
from jax.experimental import pallas as pl
from jax.experimental.pallas import tpu as pltpu
from jax.experimental.pallas import tpu_sc as plsc

_i32 = jnp.int32
_MIN_NORM = 1e-15
_EPS = 1e-7
_N = 10000
_E = 320000
_D = 128

_CHUNK = 128
_CORES = 2
_SUBCORES = 16
_TILES = _CORES * _SUBCORES
_CHUNKS = 80
_NPAD = 10240
_BR = 2000


def _arcosh(t):
    return jnp.log(t + jnp.sqrt(jnp.clip(t * t - 1.0, 0.0, None)))


def _tangent_body(x_ref, xt_ref):
    xb = x_ref[...]
    col = lax.broadcasted_iota(jnp.int32, xb.shape, 1)
    y = jnp.where(col == 0, 0.0, xb)
    n = jnp.maximum(jnp.sqrt(jnp.sum(y * y, axis=1, keepdims=True)), _MIN_NORM)
    e = jnp.exp(n)
    rest = (0.5 * (e - 1.0 / e)) * y / n
    r2 = jnp.sum(rest * rest, axis=1, keepdims=True)
    first = jnp.sqrt(jnp.maximum(1.0 + r2, _EPS))
    yn = jnp.maximum(jnp.sqrt(r2), _MIN_NORM)
    theta = jnp.maximum(first, 1.0 + _EPS)
    xt_ref[...] = jnp.where(col == 0, 0.0, (_arcosh(theta) / yn) * rest)


def _combine_body(eps_ref, x_ref, xt_ref, p_ref, w1_ref, b1_ref, w2_ref, b2_ref,
                  o_ref):
    eps = eps_ref[0]
    xb = x_ref[...]
    xt = xt_ref[...]
    sup = p_ref[0] + p_ref[1]
    col = lax.broadcasted_iota(jnp.int32, xb.shape, 1)

    def c0(a):
        return jnp.sum(jnp.where(col == 0, a, 0.0), axis=1, keepdims=True)

    def mdot(a, b):
        return jnp.sum(a * b, axis=1, keepdims=True) - 2.0 * c0(a) * c0(b)

    def expmap0(u):
        y = jnp.where(col == 0, 0.0, u)
        n = jnp.maximum(jnp.sqrt(jnp.sum(y * y, axis=1, keepdims=True)), _MIN_NORM)
        e = jnp.exp(n)
        rest = (0.5 * (e - 1.0 / e)) * y / n
        r2 = jnp.sum(rest * rest, axis=1, keepdims=True)
        first = jnp.sqrt(jnp.maximum(1.0 + r2, _EPS))
        return jnp.where(col == 0, first, rest)

    def logmap0(xh):
        y = jnp.where(col == 0, 0.0, xh)
        yn = jnp.maximum(jnp.sqrt(jnp.sum(y * y, axis=1, keepdims=True)), _MIN_NORM)
        theta = jnp.maximum(c0(xh), 1.0 + _EPS)
        return jnp.where(col == 0, 0.0, (_arcosh(theta) / yn) * y)

    x_h = expmap0(xb)
    out1 = expmap0(sup)
    log_out = logmap0(out1)

    prod = mdot(x_h, out1)
    theta_d = jnp.maximum(-prod, 1.0 + _EPS)
    sq = jnp.minimum(_arcosh(theta_d) ** 2, 50.0)
    dist = jnp.sqrt(sq)

    def logmap_pair(a, b):
        xy = jnp.minimum(mdot(a, b) + 1.0, -_EPS) - 1.0
        u = b + xy * a
        normu = jnp.maximum(jnp.sqrt(jnp.maximum(mdot(u, u), _EPS)), _MIN_NORM)
        return dist * u / normu

    logxy = logmap_pair(x_h, out1)
    logyx = logmap_pair(out1, x_h)
    alpha = mdot(logxy, xt) / jnp.maximum(sq, _MIN_NORM)
    res = xt - alpha * (logxy + logyx)
    ux = jnp.sum(jnp.where(col == 0, 0.0, out1 * res), axis=1, keepdims=True)
    first_pt = ux / jnp.maximum(c0(out1), _EPS)
    pt = jnp.where(col == 0, first_pt, res)

    out2 = expmap0(log_out + (1.0 + eps) * pt)
    h = jnp.maximum(jnp.dot(out2, w1_ref[...],
                            preferred_element_type=jnp.float32) + b1_ref[...], 0.0)
    o_ref[...] = jnp.dot(h, w2_ref[...],
                         preferred_element_type=jnp.float32) + b2_ref[...]


def _sc_segment_sum(xt, src_blocks, dst_blocks, zeros):
    mesh = plsc.VectorSubcoreMesh(core_axis_name="c", subcore_axis_name="s")
    grp = 8

    @functools.partial(
        pl.kernel,
        mesh=mesh,
        out_type=jax.ShapeDtypeStruct((_CORES, _NPAD, _D), jnp.float32),
        scratch_types=[
            pltpu.VMEM((grp, _CHUNK), jnp.int32),
            pltpu.VMEM((grp, _CHUNK), jnp.int32),
            pltpu.VMEM((_CHUNK, _D), jnp.float32),
            pltpu.VMEM((_CHUNK, _D), jnp.float32),
            pltpu.VMEM_SHARED((_NPAD, _D), jnp.float32),
            pltpu.SemaphoreType.DMA,
            pltpu.SemaphoreType.DMA,
        ],
    )
    def body(xt_hbm, src_hbm, dst_hbm, zeros_hbm, out_hbm,
             sidx, didx, buf0, buf1, acc, sem0, sem1):
        cid = lax.axis_index("c")
        sid = lax.axis_index("s")
        rz = _NPAD // _SUBCORES
        pltpu.sync_copy(zeros_hbm, acc.at[pl.ds(sid * rz, rz)])
        plsc.subcore_barrier()

        bufs = (buf0, buf1)
        sems = (sem0, sem1)

        def gather(k, p):
            return pltpu.make_async_copy(xt_hbm.at[sidx.at[_i32(k)]],
                                         bufs[p], sems[p])

        def step(i, carry):
            j0 = pl.multiple_of(i * jnp.int32(grp), grp)
            pltpu.sync_copy(src_hbm.at[cid, sid, pl.ds(j0, grp)], sidx)
            pltpu.sync_copy(dst_hbm.at[cid, sid, pl.ds(j0, grp)], didx)
            gather(0, 0).start()
            gather(1, 1).start()
            for k in range(grp):
                p = k % 2
                gather(k, p).wait()
                pltpu.sync_copy(bufs[p], acc.at[didx.at[_i32(k)]], add=True)
                if k + 2 < grp:
                    gather(k + 2, p).start()
            return carry

        lax.fori_loop(jnp.int32(0), jnp.int32(_CHUNKS // grp), step,
                      jnp.int32(0))
        plsc.subcore_barrier()
        pltpu.sync_copy(acc.at[pl.ds(sid * rz, rz)],
                        out_hbm.at[cid, pl.ds(sid * rz, rz)])

    return body(xt, src_blocks, dst_blocks, zeros)


def kernel(x, edge_index, eps, W1, b1, W2, b2):
    x = x.astype(jnp.float32)
    dst = edge_index[0].astype(jnp.int32)
    src = edge_index[1].astype(jnp.int32)

    xt = pl.pallas_call(
        _tangent_body,
        grid=(_N // _BR,),
        in_specs=[pl.BlockSpec((_BR, _D), lambda i: (_i32(i), _i32(0)))],
        out_specs=pl.BlockSpec((_BR, _D), lambda i: (_i32(i), _i32(0))),
        out_shape=jax.ShapeDtypeStruct((_N, _D), jnp.float32),
    )(x)

    pad_n = _TILES * _CHUNKS * _CHUNK - _E
    pad_src = jnp.arange(pad_n, dtype=jnp.int32) % _N
    src_p = jnp.concatenate([src, pad_src]).reshape(
        _CORES, _SUBCORES, _CHUNKS, _CHUNK)
    pad_dst = _N + (jnp.arange(pad_n, dtype=jnp.int32) % (_NPAD - _N))
    dst_p = jnp.concatenate([dst, pad_dst]).reshape(
        _CORES, _SUBCORES, _CHUNKS, _CHUNK)
    zeros = jnp.zeros((_NPAD // _SUBCORES, _D), jnp.float32)

    partials = _sc_segment_sum(xt, src_p, dst_p, zeros)

    eps_arr = jnp.reshape(eps.astype(jnp.float32), (1,))
    out = pl.pallas_call(
        _combine_body,
        grid=(_N // _BR,),
        in_specs=[
            pl.BlockSpec((1,), lambda i: (_i32(0),), memory_space=pltpu.SMEM),
            pl.BlockSpec((_BR, _D), lambda i: (_i32(i), _i32(0))),
            pl.BlockSpec((_BR, _D), lambda i: (_i32(i), _i32(0))),
            pl.BlockSpec((_CORES, _BR, _D), lambda i: (_i32(0), _i32(i), _i32(0))),
            pl.BlockSpec((_D, _D), lambda i: (_i32(0), _i32(0))),
            pl.BlockSpec((1, _D), lambda i: (_i32(0), _i32(0))),
            pl.BlockSpec((_D, _D), lambda i: (_i32(0), _i32(0))),
            pl.BlockSpec((1, _D), lambda i: (_i32(0), _i32(0))),
        ],
        out_specs=pl.BlockSpec((_BR, _D), lambda i: (_i32(i), _i32(0))),
        out_shape=jax.ShapeDtypeStruct((_N, _D), jnp.float32),
    )(eps_arr, x, xt, partials, W1, jnp.reshape(b1, (1, _D)),
      W2, jnp.reshape(b2, (1, _D)))
    return out
